# baseline (speedup 1.0000x reference)
import functools

import jax
import jax.numpy as jnp
from jax import lax
from jax.experimental import pallas as pl
from jax.experimental.pallas import tpu as pltpu
from jax.experimental.pallas import tpu_sc as plsc

N_TOTAL = 100000
N_SRC_L1 = 10000
N_LIVE = 2048
N_DST2 = 2048
E1 = 320000
E2 = 65536
D_IN = 128
D_OUT = 128

XBLK = 4096
NPAD = 25 * XBLK

NGATHER = 10240
NT = 16
GY_PER_T = NGATHER // NT
GY_CHUNK = 128
GY_NCHUNK = GY_PER_T // GY_CHUNK

E1_PER_T = E1 // NT
E1_CHUNK = 2000
E1_UNROLL = 5
E2_PER_T = E2 // NT
E2_CHUNK = 2048
E2_UNROLL = 4
SLICE = N_LIVE // NT
OUT_PER_T = N_DST2 // NT

_SC_PARAMS = pltpu.CompilerParams(needs_layout_passes=False,
                                  use_tc_tiling_on_sc=False)


def _project_kernel(x_ref, w_ref, z_ref):
    i = pl.program_id(0)
    z_ref[:, pl.ds(i * XBLK, XBLK)] = lax.dot_general(
        w_ref[...], x_ref[...], (((0,), (1,)), ((), ())),
        preferred_element_type=jnp.float32)


def _msg_kernel(z_hbm, nid_hbm, s1_hbm, d1_hbm, s2_hbm, d2_hbm,
                b1_hbm, w2_hbm, b2_hbm, out_hbm,
                zidx_v, cidx_v, zrows_v, y_v, e1s_v, e1d_v, agg_v, h_v,
                e2s_v, e2d_v,
                b1_v, w2_v, b2_v, pstage_v, pacc_v, mloc_v, outbuf_v,
                y_s, parts_s, h_s, sem):
    t = lax.axis_index("s")
    ones = jnp.full((16,), 1.0, jnp.float32)
    csplat = [jnp.full((16,), c, jnp.int32) for c in range(4)]

    pltpu.sync_copy(nid_hbm.at[pl.ds(t * GY_PER_T, GY_PER_T)], zidx_v)

    def _mkidx(k, _):
        sl = pl.ds(k * 16, 16)
        base = zidx_v[sl]
        for c in range(3):
            cidx_v[c, sl] = base + jnp.full((16,), c * N_TOTAL, jnp.int32)
        return 0
    lax.fori_loop(0, GY_PER_T // 16, _mkidx, 0)
    copies = [
        pltpu.async_copy(
            z_hbm.at[cidx_v.at[c, pl.ds(j * GY_CHUNK, GY_CHUNK)]],
            zrows_v.at[c, j], sem)
        for c in range(3) for j in range(GY_NCHUNK)
    ]
    pltpu.sync_copy(b1_hbm, b1_v)
    pltpu.sync_copy(w2_hbm, w2_v)
    pltpu.sync_copy(b2_hbm, b2_v)

    def _zpriv(i, _):
        z = jnp.zeros((16,), jnp.float32)
        sl = pl.ds(i * 16, 16)
        for c in range(4):
            agg_v[c, sl] = z
        return 0
    lax.fori_loop(0, N_LIVE // 16, _zpriv, 0)

    for c in range(3):
        for j in range(GY_NCHUNK):
            copies[c * GY_NCHUNK + j].wait()
            pltpu.sync_copy(
                zrows_v.at[c, j],
                y_s.at[c, pl.ds(t * GY_PER_T + j * GY_CHUNK, GY_CHUNK)])
    plsc.subcore_barrier()
    pltpu.sync_copy(y_s, y_v)

    for chunk in range(E1_PER_T // E1_CHUNK):
        base = t * E1_PER_T + chunk * E1_CHUNK
        pltpu.sync_copy(s1_hbm.at[pl.ds(base, E1_CHUNK)], e1s_v)
        pltpu.sync_copy(d1_hbm.at[pl.ds(base, E1_CHUNK)], e1d_v)

        def _edge1(i, _):
            for u in range(E1_UNROLL):
                sl = pl.ds(i * (16 * E1_UNROLL) + u * 16, 16)
                s16 = e1s_v[sl]
                d16 = e1d_v[sl]
                live = d16 < N_LIVE
                dc = jnp.minimum(d16, N_LIVE - 1)
                for c in range(3):
                    v = plsc.load_gather(y_v, [csplat[c], s16])
                    plsc.addupdate_scatter(agg_v, [csplat[c], dc], v,
                                           mask=live)
                plsc.addupdate_scatter(agg_v, [csplat[3], dc], ones,
                                       mask=live)
            return 0
        lax.fori_loop(0, E1_CHUNK // (16 * E1_UNROLL), _edge1, 0)

    pltpu.sync_copy(agg_v, parts_s.at[t])
    plsc.subcore_barrier()

    csl = pl.ds(t * SLICE, SLICE)

    def _zacc(i, _):
        sl = pl.ds(i * 16, 16)
        for c in range(4):
            pacc_v[c, sl] = jnp.zeros((16,), jnp.float32)
        return 0
    lax.fori_loop(0, SLICE // 16, _zacc, 0)
    for p in range(NT):
        pltpu.sync_copy(parts_s.at[p, :, csl], pstage_v)

        def _accp(i, _):
            sl = pl.ds(i * 16, 16)
            for c in range(4):
                pacc_v[c, sl] = pacc_v[c, sl] + pstage_v[c, sl]
            return 0
        lax.fori_loop(0, SLICE // 16, _accp, 0)

    def _hrow(i, _):
        sl = pl.ds(i * 16, 16)
        den = jnp.maximum(pacc_v[3, sl], 1.0)
        for c in range(3):
            pstage_v[c, sl] = jnp.maximum(
                pacc_v[c, sl] / den + b1_v[pl.ds(c * 16, 16)], 0.0)
        return 0
    lax.fori_loop(0, SLICE // 16, _hrow, 0)
    pltpu.sync_copy(pstage_v, h_s.at[:, csl])
    plsc.subcore_barrier()

    pltpu.sync_copy(h_s, h_v)
    lax.fori_loop(0, N_LIVE // 16, _zpriv, 0)
    for chunk2 in range(E2_PER_T // E2_CHUNK):
        base2 = t * E2_PER_T + chunk2 * E2_CHUNK
        pltpu.sync_copy(s2_hbm.at[pl.ds(base2, E2_CHUNK)], e2s_v)
        pltpu.sync_copy(d2_hbm.at[pl.ds(base2, E2_CHUNK)], e2d_v)

        def _edge2(i, _):
            for u in range(E2_UNROLL):
                sl = pl.ds(i * (16 * E2_UNROLL) + u * 16, 16)
                s16 = e2s_v[sl]
                d16 = e2d_v[sl]
                for c in range(3):
                    v = plsc.load_gather(h_v, [csplat[c], s16])
                    plsc.addupdate_scatter(agg_v, [csplat[c], d16], v)
                plsc.addupdate_scatter(agg_v, [csplat[3], d16], ones)
            return 0
        lax.fori_loop(0, E2_CHUNK // (16 * E2_UNROLL), _edge2, 0)

    pltpu.sync_copy(agg_v, parts_s.at[t])
    plsc.subcore_barrier()

    lax.fori_loop(0, SLICE // 16, _zacc, 0)
    for p in range(NT):
        pltpu.sync_copy(parts_s.at[p, :, csl], pstage_v)

        def _accp2(i, _):
            sl = pl.ds(i * 16, 16)
            for c in range(4):
                pacc_v[c, sl] = pacc_v[c, sl] + pstage_v[c, sl]
            return 0
        lax.fori_loop(0, SLICE // 16, _accp2, 0)

    def _mrow(i, _):
        sl = pl.ds(i * 16, 16)
        den = jnp.maximum(pacc_v[3, sl], 1.0)
        for c in range(3):
            mloc_v[c, sl] = pacc_v[c, sl] / den
        return 0
    lax.fori_loop(0, OUT_PER_T // 16, _mrow, 0)

    b2c = [b2_v[pl.ds(j * 16, 16)] for j in range(8)]
    w2c = [[w2_v[c, pl.ds(j * 16, 16)] for j in range(8)] for c in range(3)]

    for quarter in range(4):
        def _orow(d, _, quarter=quarter):
            dd = jnp.full((16,), quarter * 32, jnp.int32) + d
            mb = [plsc.load_gather(mloc_v, [csplat[c], dd]) for c in range(3)]
            for j in range(8):
                acc = (b2c[j] + mb[0] * w2c[0][j] + mb[1] * w2c[1][j]
                       + mb[2] * w2c[2][j])
                outbuf_v[d, pl.ds(j * 16, 16)] = acc
            return 0
        lax.fori_loop(0, OUT_PER_T // 4, _orow, 0)
        pltpu.sync_copy(
            outbuf_v,
            out_hbm.at[pl.ds(t * OUT_PER_T + quarter * 32, OUT_PER_T // 4)])


_msg = functools.partial(
    pl.kernel,
    out_type=jax.ShapeDtypeStruct((N_DST2, D_OUT), jnp.float32),
    mesh=plsc.VectorSubcoreMesh(core_axis_name="c", subcore_axis_name="s",
                                num_cores=1, num_subcores=NT),
    scratch_types=[
        pltpu.VMEM((GY_PER_T,), jnp.int32),
        pltpu.VMEM((3, GY_PER_T), jnp.int32),
        pltpu.VMEM((3, GY_NCHUNK, GY_CHUNK), jnp.float32),
        pltpu.VMEM((3, NGATHER), jnp.float32),
        pltpu.VMEM((E1_CHUNK,), jnp.int32),
        pltpu.VMEM((E1_CHUNK,), jnp.int32),
        pltpu.VMEM((4, N_LIVE), jnp.float32),
        pltpu.VMEM((4, N_LIVE), jnp.float32),
        pltpu.VMEM((E2_CHUNK,), jnp.int32),
        pltpu.VMEM((E2_CHUNK,), jnp.int32),
        pltpu.VMEM((128,), jnp.float32),
        pltpu.VMEM((3, D_OUT), jnp.float32),
        pltpu.VMEM((D_OUT,), jnp.float32),
        pltpu.VMEM((4, SLICE), jnp.float32),
        pltpu.VMEM((4, SLICE), jnp.float32),
        pltpu.VMEM((3, OUT_PER_T), jnp.float32),
        pltpu.VMEM((OUT_PER_T // 4, D_OUT), jnp.float32),
        pltpu.VMEM_SHARED((3, NGATHER), jnp.float32),
        pltpu.VMEM_SHARED((NT, 4, N_LIVE), jnp.float32),
        pltpu.VMEM_SHARED((4, N_LIVE), jnp.float32),
        pltpu.SemaphoreType.DMA,
    ],
    compiler_params=_SC_PARAMS,
)(_msg_kernel)


def kernel(x, n_id, edge_index1, edge_index2, W1, b1, W2, b2):
    nid = n_id[:N_SRC_L1].astype(jnp.int32)
    nid = jnp.pad(nid, (0, NGATHER - N_SRC_L1))
    s1 = edge_index1[0].astype(jnp.int32)
    d1 = edge_index1[1].astype(jnp.int32)
    s2 = edge_index2[0].astype(jnp.int32)
    d2 = edge_index2[1].astype(jnp.int32)
    w1p = jnp.pad(W1, ((0, 0), (0, 5)))
    b1s = jnp.pad(jnp.repeat(b1, 16), (0, 128 - 48))

    zt = pl.pallas_call(
        _project_kernel,
        grid=(NPAD // XBLK,),
        in_specs=[
            pl.BlockSpec((XBLK, D_IN), lambda i: (i, 0)),
            pl.BlockSpec((D_IN, 8), lambda i: (0, 0)),
        ],
        out_specs=pl.BlockSpec((8, NPAD), lambda i: (0, 0)),
        out_shape=jax.ShapeDtypeStruct((8, NPAD), jnp.float32),
    )(x, w1p)
    z1d = zt[:3, :N_TOTAL].reshape(3 * N_TOTAL)
    return _msg(z1d, nid, s1, d1, s2, d2, b1s, W2, b2)


# speedup vs baseline: 1.3043x; 1.3043x over previous
import functools

import jax
import jax.numpy as jnp
from jax import lax
from jax.experimental import pallas as pl
from jax.experimental.pallas import tpu as pltpu
from jax.experimental.pallas import tpu_sc as plsc

N_SRC_L1 = 10000
N_LIVE = 2048
N_DST2 = 2048
E1 = 320000
E2 = 65536
D_IN = 128
D_OUT = 128

NGATHER = 10240
GW = 32
G_PER_W = NGATHER // GW
G_CHUNK = 64
G_NCHUNK = G_PER_W // G_CHUNK

NT = 16
E1_PER_T = E1 // NT
E1_CHUNK = 4000
E1_UNROLL = 5
E2_PER_T = E2 // NT
E2_UNROLL = 4
SLICE = N_LIVE // NT
OUT_PER_T = N_DST2 // NT

_SC_PARAMS = pltpu.CompilerParams(needs_layout_passes=False)


def _gather_kernel(x_hbm, nid_hbm, g_hbm, idx_v, rows_v, sem, osem):
    wid = lax.axis_index("s") * 2 + lax.axis_index("c")
    pltpu.sync_copy(nid_hbm.at[wid], idx_v)
    copies = [
        pltpu.async_copy(x_hbm.at[idx_v.at[j]], rows_v.at[j], sem)
        for j in range(G_NCHUNK)
    ]
    outs = []
    for j in range(G_NCHUNK):
        copies[j].wait()
        outs.append(pltpu.async_copy(
            rows_v.at[j],
            g_hbm.at[pl.ds(wid * G_PER_W + j * G_CHUNK, G_CHUNK)], osem))
    for o in outs:
        o.wait()


_gather = functools.partial(
    pl.kernel,
    out_type=jax.ShapeDtypeStruct((NGATHER, D_IN), jnp.float32),
    mesh=plsc.VectorSubcoreMesh(core_axis_name="c", subcore_axis_name="s"),
    scratch_types=[
        pltpu.VMEM((G_NCHUNK, G_CHUNK), jnp.int32),
        pltpu.VMEM((G_NCHUNK, G_CHUNK, D_IN), jnp.float32),
        pltpu.SemaphoreType.DMA,
        pltpu.SemaphoreType.DMA,
    ],
    compiler_params=_SC_PARAMS,
)(_gather_kernel)


def _project_kernel(g_ref, w_ref, yt_ref):
    yt_ref[...] = lax.dot_general(
        w_ref[...], g_ref[...], (((0,), (1,)), ((), ())),
        preferred_element_type=jnp.float32)


def _msg_kernel(yt_hbm, s1_hbm, d1_hbm, s2_hbm, d2_hbm,
                b1_hbm, w2_hbm, b2_hbm, out_hbm,
                y_v, e1s_v, e1d_v, agg_v, h_v, e2s_v, e2d_v,
                b1_v, w2_v, b2_v, pall_v, pstage_v, pacc_v, mloc_v, outbuf_v,
                parts_s, h_s):
    t = lax.axis_index("s")
    ones = jnp.full((16,), 1.0, jnp.float32)
    csplat = [jnp.full((16,), c, jnp.int32) for c in range(4)]

    pltpu.sync_copy(yt_hbm, y_v)
    pltpu.sync_copy(b1_hbm, b1_v)
    pltpu.sync_copy(w2_hbm, w2_v)
    pltpu.sync_copy(b2_hbm, b2_v)

    def _zpriv(i, _):
        z = jnp.zeros((16,), jnp.float32)
        sl = pl.ds(i * 16, 16)
        for c in range(4):
            agg_v[c, sl] = z
        return 0
    lax.fori_loop(0, N_LIVE // 16, _zpriv, 0)

    for chunk in range(E1_PER_T // E1_CHUNK):
        base = t * E1_PER_T + chunk * E1_CHUNK
        pltpu.sync_copy(s1_hbm.at[pl.ds(base, E1_CHUNK)], e1s_v)
        pltpu.sync_copy(d1_hbm.at[pl.ds(base, E1_CHUNK)], e1d_v)

        def _edge1(i, _):
            for u in range(E1_UNROLL):
                sl = pl.ds(i * (16 * E1_UNROLL) + u * 16, 16)
                s16 = e1s_v[sl]
                d16 = e1d_v[sl]
                live = d16 < N_LIVE
                dc = jnp.minimum(d16, N_LIVE - 1)
                for c in range(3):
                    v = plsc.load_gather(y_v, [csplat[c], s16])
                    plsc.addupdate_scatter(agg_v, [csplat[c], dc], v,
                                           mask=live)
                plsc.addupdate_scatter(agg_v, [csplat[3], dc], ones,
                                       mask=live)
            return 0
        lax.fori_loop(0, E1_CHUNK // (16 * E1_UNROLL), _edge1, 0)

    pltpu.sync_copy(agg_v, parts_s.at[t])
    plsc.subcore_barrier()

    csl = pl.ds(t * SLICE, SLICE)

    def _reduce_slots():
        pltpu.sync_copy(parts_s.at[:, :, csl], pall_v)

        def _acc0(i, _):
            sl = pl.ds(i * 16, 16)
            for c in range(4):
                pacc_v[c, sl] = pall_v[0, c, sl]
            return 0
        lax.fori_loop(0, SLICE // 16, _acc0, 0)
        for p in range(1, NT):
            def _accp(i, _, p=p):
                sl = pl.ds(i * 16, 16)
                for c in range(4):
                    pacc_v[c, sl] = pacc_v[c, sl] + pall_v[p, c, sl]
                return 0
            lax.fori_loop(0, SLICE // 16, _accp, 0)

    _reduce_slots()

    def _hrow(i, _):
        sl = pl.ds(i * 16, 16)
        den = jnp.maximum(pacc_v[3, sl], 1.0)
        for c in range(3):
            pstage_v[c, sl] = jnp.maximum(
                pacc_v[c, sl] / den + b1_v[c, :], 0.0)
        return 0
    lax.fori_loop(0, SLICE // 16, _hrow, 0)
    pltpu.sync_copy(pstage_v, h_s.at[:, csl])
    plsc.subcore_barrier()

    pltpu.sync_copy(h_s, h_v)
    lax.fori_loop(0, N_LIVE // 16, _zpriv, 0)
    base2 = t * E2_PER_T
    pltpu.sync_copy(s2_hbm.at[pl.ds(base2, E2_PER_T)], e2s_v)
    pltpu.sync_copy(d2_hbm.at[pl.ds(base2, E2_PER_T)], e2d_v)

    def _edge2(i, _):
        for u in range(E2_UNROLL):
            sl = pl.ds(i * (16 * E2_UNROLL) + u * 16, 16)
            s16 = e2s_v[sl]
            d16 = e2d_v[sl]
            for c in range(3):
                v = plsc.load_gather(h_v, [csplat[c], s16])
                plsc.addupdate_scatter(agg_v, [csplat[c], d16], v)
            plsc.addupdate_scatter(agg_v, [csplat[3], d16], ones)
        return 0
    lax.fori_loop(0, E2_PER_T // (16 * E2_UNROLL), _edge2, 0)

    pltpu.sync_copy(agg_v, parts_s.at[t])
    plsc.subcore_barrier()

    _reduce_slots()

    def _mrow(i, _):
        sl = pl.ds(i * 16, 16)
        den = jnp.maximum(pacc_v[3, sl], 1.0)
        for c in range(3):
            mloc_v[c, sl] = pacc_v[c, sl] / den
        return 0
    lax.fori_loop(0, OUT_PER_T // 16, _mrow, 0)

    b2c = [b2_v[pl.ds(j * 16, 16)] for j in range(8)]
    w2c = [[w2_v[c, pl.ds(j * 16, 16)] for j in range(8)] for c in range(3)]

    def _orow(d, _):
        dd = jnp.full((16,), 0, jnp.int32) + d
        mb = [plsc.load_gather(mloc_v, [csplat[c], dd]) for c in range(3)]
        for j in range(8):
            acc = (b2c[j] + mb[0] * w2c[0][j] + mb[1] * w2c[1][j]
                   + mb[2] * w2c[2][j])
            outbuf_v[d, pl.ds(j * 16, 16)] = acc
        return 0
    lax.fori_loop(0, OUT_PER_T, _orow, 0)
    pltpu.sync_copy(outbuf_v, out_hbm.at[pl.ds(t * OUT_PER_T, OUT_PER_T)])


_msg = functools.partial(
    pl.kernel,
    out_type=jax.ShapeDtypeStruct((N_DST2, D_OUT), jnp.float32),
    mesh=plsc.VectorSubcoreMesh(core_axis_name="c", subcore_axis_name="s",
                                num_cores=1, num_subcores=NT),
    scratch_types=[
        pltpu.VMEM((4, NGATHER), jnp.float32),
        pltpu.VMEM((E1_CHUNK,), jnp.int32),
        pltpu.VMEM((E1_CHUNK,), jnp.int32),
        pltpu.VMEM((4, N_LIVE), jnp.float32),
        pltpu.VMEM((4, N_LIVE), jnp.float32),
        pltpu.VMEM((E2_PER_T,), jnp.int32),
        pltpu.VMEM((E2_PER_T,), jnp.int32),
        pltpu.VMEM((3, 16), jnp.float32),
        pltpu.VMEM((3, D_OUT), jnp.float32),
        pltpu.VMEM((D_OUT,), jnp.float32),
        pltpu.VMEM((NT, 4, SLICE), jnp.float32),
        pltpu.VMEM((4, SLICE), jnp.float32),
        pltpu.VMEM((4, SLICE), jnp.float32),
        pltpu.VMEM((3, OUT_PER_T), jnp.float32),
        pltpu.VMEM((OUT_PER_T, D_OUT), jnp.float32),
        pltpu.VMEM_SHARED((NT, 4, N_LIVE), jnp.float32),
        pltpu.VMEM_SHARED((4, N_LIVE), jnp.float32),
    ],
    compiler_params=_SC_PARAMS,
)(_msg_kernel)


def kernel(x, n_id, edge_index1, edge_index2, W1, b1, W2, b2):
    nid = n_id[:N_SRC_L1].astype(jnp.int32)
    nid = jnp.pad(nid, (0, NGATHER - N_SRC_L1)).reshape(GW, G_NCHUNK, G_CHUNK)
    s1 = edge_index1[0].astype(jnp.int32)
    d1 = edge_index1[1].astype(jnp.int32)
    s2 = edge_index2[0].astype(jnp.int32)
    d2 = edge_index2[1].astype(jnp.int32)
    w1p = jnp.pad(W1, ((0, 0), (0, 1)))
    b1s = jnp.tile(b1[:, None], (1, 16))

    g = _gather(x, nid)
    yt = pl.pallas_call(
        _project_kernel,
        out_shape=jax.ShapeDtypeStruct((4, NGATHER), jnp.float32),
    )(g, w1p)
    return _msg(yt, s1, d1, s2, d2, b1s, W2, b2)
